# Initial kernel scaffold; baseline (speedup 1.0000x reference)
import jax
import jax.numpy as jnp
from jax.experimental import pallas as pl


def kernel(decoder_out_att, ys_out_pad_att):
    raise NotImplementedError("write your pallas kernel here")


# speedup vs baseline: 1.5626x; 1.5626x over previous
import jax
import jax.numpy as jnp
from jax.experimental import pallas as pl
from jax.experimental.pallas import tpu as pltpu

_BINS = 100
_IGNORE = 0
_R = 8


def _stat_block(x_ref, ys_ref, acc_ref):
    i = pl.program_id(0)

    @pl.when(i == 0)
    def _():
        acc_ref[...] = jnp.zeros_like(acc_ref)

    x = x_ref[...]
    ys = ys_ref[0, 0, :]
    col = jax.lax.broadcasted_iota(jnp.int32, x.shape, 1)
    tgt = jnp.sum(jnp.where(col == ys[:, None], x, 0.0), axis=1)
    m = jnp.max(x, axis=1)
    s = jnp.sum(jnp.exp(x - m[:, None]), axis=1)
    pv = jnp.exp(tgt - m) / s
    valid = (ys != _IGNORE).astype(jnp.float32)

    lanes_i = jax.lax.broadcasted_iota(jnp.int32, (x.shape[0], 128), 1)
    lanes_f = lanes_i.astype(jnp.float32)
    upper = pv[:, None] > lanes_f / _BINS
    lower = pv[:, None] < lanes_f + (1.0 / _BINS)
    mask = (upper & lower & (lanes_i < _BINS)).astype(jnp.float32) * valid[:, None]
    hist = jnp.sum(mask, axis=0)

    lane1 = jax.lax.iota(jnp.int32, 128)
    extra = jnp.where(lane1 == _BINS, jnp.sum(pv * valid),
                      jnp.where(lane1 == _BINS + 1, jnp.sum(valid), 0.0))
    acc_ref[0, :] += hist + extra


def kernel(decoder_out_att, ys_out_pad_att):
    B, T, V = decoder_out_att.shape
    N = B * T
    x = decoder_out_att.reshape(N, V)
    ys = ys_out_pad_att.reshape(N // _R, 1, _R)
    acc = pl.pallas_call(
        _stat_block,
        grid=(N // _R,),
        in_specs=[pl.BlockSpec((_R, V), lambda i: (i, 0)),
                  pl.BlockSpec((1, 1, _R), lambda i: (i, 0, 0))],
        out_specs=pl.BlockSpec((1, 128), lambda i: (0, 0)),
        out_shape=jax.ShapeDtypeStruct((1, 128), jnp.float32),
        compiler_params=pltpu.CompilerParams(dimension_semantics=("arbitrary",)),
    )(x, ys)[0]
    mean = acc[_BINS] / jnp.maximum(acc[_BINS + 1], 1.0)
    return jnp.concatenate([mean[None], acc[:_BINS]])
